# baseline (speedup 1.0000x reference)
import functools

import jax
import jax.numpy as jnp
from jax import lax
from jax.experimental import pallas as pl
from jax.experimental.pallas import tpu as pltpu
from jax.experimental.pallas import tpu_sc as plsc

N = 10000
NG = 100
K = 16
E = N * K

KNN_R = 200
KNN_C = 512
NPAD = 10240

EB = 6400
NB = 400
NSTEP = E // EB

XB = 2000
XSTEP = N // XB

_F32 = jnp.float32
_I32 = jnp.int32
_INF = float("inf")


def _dot(a, b):
    return lax.dot_general(a, b, (((1,), (0,)), ((), ())),
                           preferred_element_type=_F32)


def _knn_body(xp_ref, bp_ref, brow_ref, bsm_ref, out_ref):
    t = pl.program_id(0)
    rs = t * KNN_R
    xr = xp_ref[pl.ds(rs, KNN_R), :]
    rb = brow_ref[pl.ds(rs, KNN_R), :]
    sqr = jnp.sum(xr * xr, axis=1, keepdims=True)

    bp = bp_ref[...]
    b0 = bsm_ref[rs]
    bl = bsm_ref[rs + KNN_R - 1]
    wstart = jnp.sum((bp < b0).astype(_I32))
    wend = jnp.sum((bp <= bl).astype(_I32))
    ws8 = jnp.minimum((wstart // 8) * 8, NPAD - KNN_C)

    ci = lax.broadcasted_iota(_I32, (KNN_R, KNN_C), 1)
    ri = rs + lax.broadcasted_iota(_I32, (KNN_R, KNN_C), 0)
    hc = KNN_C // 2
    big = 1 << 30

    def chunk(carry):
        k, tv, ti = carry
        cs = ws8 + k * KNN_C
        xc = xp_ref[pl.ds(cs, KNN_C), :]
        cbt = brow_ref[pl.ds(cs, KNN_C), :].reshape(1, KNN_C)
        sqc = jnp.sum(xc * xc, axis=1, keepdims=True).reshape(1, KNN_C)
        dots = lax.dot_general(xr, xc, (((1,), (1,)), ((), ())),
                               preferred_element_type=_F32)
        d2 = sqr + sqc - 2.0 * dots
        gci = ci + cs
        bad = (rb != cbt) | (ri == gci)
        d2 = jnp.where(bad, _INF, d2)

        a, b = d2[:, :hc], d2[:, hc:]
        ia, ib = gci[:, :hc], gci[:, hc:]
        sel = a <= b
        lo = jnp.where(sel, a, b)
        hi = jnp.where(sel, b, a)
        li = jnp.where(sel, ia, ib)
        hj = jnp.where(sel, ib, ia)

        cv = jnp.concatenate([tv, lo], axis=1)
        cidx = jnp.concatenate([ti, li], axis=1)
        sib = jnp.concatenate([jnp.full((KNN_R, K), _INF, _F32), hi], axis=1)
        sidx = jnp.concatenate([ti, hj], axis=1)
        vs, js = [], []
        for _ in range(K):
            m = jnp.min(cv, axis=1, keepdims=True)
            elig = cv == m
            idxk = jnp.min(jnp.where(elig, cidx, big), axis=1, keepdims=True)
            kill = elig & (cidx == idxk)
            vs.append(m)
            js.append(idxk)
            cv = jnp.where(kill, sib, cv)
            cidx = jnp.where(kill, sidx, cidx)
            sib = jnp.where(kill, _INF, sib)
        tv = jnp.concatenate(vs, axis=1)
        ti = jnp.concatenate(js, axis=1)
        return k + 1, tv, ti

    def cond(carry):
        return ws8 + carry[0] * KNN_C < wend

    tv0 = jnp.full((KNN_R, K), _INF, _F32)
    ti0 = jnp.zeros((KNN_R, K), _I32)
    _, _, ti = lax.while_loop(cond, chunk, (0, tv0, ti0))
    out_ref[...] = ti


def _knn(x, bp, brow, bsm, f):
    return pl.pallas_call(
        _knn_body,
        grid=(N // KNN_R,),
        in_specs=[
            pl.BlockSpec((NPAD, f), lambda t: (0, 0)),
            pl.BlockSpec((1, NPAD), lambda t: (0, 0)),
            pl.BlockSpec((NPAD, 1), lambda t: (0, 0)),
            pl.BlockSpec(memory_space=pltpu.SMEM),
        ],
        out_specs=pl.BlockSpec((KNN_R, K), lambda t: (t, 0)),
        out_shape=jax.ShapeDtypeStruct((N, K), _I32),
    )(x, bp, brow, bsm)


def _dense_body(f, cv, x_ref, w0_ref, b0_ref, scw_ref,
                u_ref, v_ref, s_ref, st_ref, acc):
    t = pl.program_id(0)
    xb = x_ref[...]
    w0 = w0_ref[...]
    wa = w0[:f, :]
    wb = w0[f:, :]
    c = w0.shape[1]
    u_ref[...] = _dot(xb, wa - wb) + b0_ref[...]
    v = _dot(xb, wb)
    if cv > c:
        v = jnp.concatenate([v, jnp.zeros((XB, cv - c), _F32)], axis=1)
    v_ref[...] = v
    s = _dot(xb, scw_ref[...])
    s_ref[...] = s

    @pl.when(t == 0)
    def _():
        acc[...] = jnp.zeros_like(acc)

    c = s.shape[1]
    upd = jnp.concatenate(
        [jnp.sum(s, 0, keepdims=True),
         jnp.sum(s * s, 0, keepdims=True),
         jnp.zeros((6, c), _F32)], axis=0)
    acc[...] += upd

    @pl.when(t == XSTEP - 1)
    def _():
        mu = acc[0:1, :] / N
        var = acc[1:2, :] / N - mu * mu
        st_ref[...] = jnp.concatenate([mu, lax.rsqrt(var + 1e-5)], axis=0)


def _dense(x, w0, b0, scw, f, c, cv):
    return pl.pallas_call(
        functools.partial(_dense_body, f, cv),
        grid=(XSTEP,),
        in_specs=[
            pl.BlockSpec((XB, f), lambda t: (t, 0)),
            pl.BlockSpec((2 * f, c), lambda t: (0, 0)),
            pl.BlockSpec((1, c), lambda t: (0, 0)),
            pl.BlockSpec((f, c), lambda t: (0, 0)),
        ],
        out_specs=[
            pl.BlockSpec((XB, c), lambda t: (t, 0)),
            pl.BlockSpec((XB, cv), lambda t: (t, 0)),
            pl.BlockSpec((XB, c), lambda t: (t, 0)),
            pl.BlockSpec((2, c), lambda t: (0, 0)),
        ],
        out_shape=[
            jax.ShapeDtypeStruct((N, c), _F32),
            jax.ShapeDtypeStruct((N, cv), _F32),
            jax.ShapeDtypeStruct((N, c), _F32),
            jax.ShapeDtypeStruct((2, c), _F32),
        ],
        scratch_shapes=[pltpu.VMEM((8, c), _F32)],
    )(x, w0, b0, scw)


_NC = 2
_NS = 16
_NW = _NC * _NS
_PERW = E // _NW
_CH = 40
_NCH = _PERW // _CH


def _edge_gather(v, idx3, c):
    mesh = plsc.VectorSubcoreMesh(core_axis_name="c", subcore_axis_name="s")

    @functools.partial(
        pl.kernel, mesh=mesh,
        out_type=jax.ShapeDtypeStruct((E, c), _F32),
        scratch_types=[
            pltpu.VMEM((_NCH, _CH), _I32),
            pltpu.VMEM((_CH, c), _F32),
            pltpu.VMEM((_CH, c), _F32),
            pltpu.SemaphoreType.DMA,
            pltpu.SemaphoreType.DMA,
        ],
    )
    def gk(v_hbm, idx_hbm, out_hbm, idx_v, buf0, buf1, sem0, sem1):
        wid = lax.axis_index("s") * _NC + lax.axis_index("c")
        base = wid * _PERW
        pltpu.sync_copy(idx_hbm.at[wid], idx_v)

        def wait0():
            pltpu.make_async_copy(v_hbm.at[idx_v.at[0]], buf0, sem0).wait()

        def wait1():
            pltpu.make_async_copy(v_hbm.at[idx_v.at[0]], buf1, sem1).wait()

        pltpu.async_copy(v_hbm.at[idx_v.at[0]], buf0, sem0)

        def step(j, carry):
            c0 = 2 * j
            pltpu.async_copy(v_hbm.at[idx_v.at[c0 + 1]], buf1, sem1)
            wait0()
            pltpu.sync_copy(buf0, out_hbm.at[pl.ds(base + c0 * _CH, _CH)])
            pltpu.async_copy(v_hbm.at[idx_v.at[c0 + 2]], buf0, sem0)
            wait1()
            pltpu.sync_copy(buf1,
                            out_hbm.at[pl.ds(base + (c0 + 1) * _CH, _CH)])
            return carry

        lax.fori_loop(0, (_NCH - 1) // 2, step, 0, unroll=False)
        wait0()
        pltpu.sync_copy(buf0,
                        out_hbm.at[pl.ds(base + (_NCH - 1) * _CH, _CH)])

    return gk(v, idx3)


def _stats0_body(vj_ref, u_ref, st_ref, acc):
    t = pl.program_id(0)
    u = u_ref[...]
    c = u.shape[1]
    urep = jnp.broadcast_to(u[:, None, :], (NB, K, c)).reshape(EB, c)
    y = vj_ref[:, :c] + urep

    @pl.when(t == 0)
    def _():
        acc[...] = jnp.zeros_like(acc)

    acc[...] += jnp.concatenate(
        [jnp.sum(y, 0, keepdims=True),
         jnp.sum(y * y, 0, keepdims=True),
         jnp.zeros((6, c), _F32)], axis=0)

    @pl.when(t == NSTEP - 1)
    def _():
        mu = acc[0:1, :] / E
        var = acc[1:2, :] / E - mu * mu
        st_ref[...] = jnp.concatenate([mu, lax.rsqrt(var + 1e-5)], axis=0)


def _stats0(vj, u, c, cv):
    return pl.pallas_call(
        _stats0_body,
        grid=(NSTEP,),
        in_specs=[
            pl.BlockSpec((EB, cv), lambda t: (t, 0)),
            pl.BlockSpec((NB, c), lambda t: (t, 0)),
        ],
        out_specs=pl.BlockSpec((2, c), lambda t: (0, 0)),
        out_shape=jax.ShapeDtypeStruct((2, c), _F32),
        scratch_shapes=[pltpu.VMEM((8, c), _F32)],
    )(vj, u)


def _mlp0_body(vj_ref, u_ref, st_ref, g_ref, be_ref, w_ref, b_ref,
               y_ref, stn_ref, acc):
    t = pl.program_id(0)
    u = u_ref[...]
    c = u.shape[1]
    urep = jnp.broadcast_to(u[:, None, :], (NB, K, c)).reshape(EB, c)
    y0 = vj_ref[:, :c] + urep
    mu = st_ref[0:1, :]
    rstd = st_ref[1:2, :]
    scale = g_ref[...] * rstd
    shift = be_ref[...] - mu * scale
    h = jnp.maximum(y0 * scale + shift, 0.0)
    y = _dot(h, w_ref[...]) + b_ref[...]
    y_ref[...] = y

    @pl.when(t == 0)
    def _():
        acc[...] = jnp.zeros_like(acc)

    cn = y.shape[1]
    acc[...] += jnp.concatenate(
        [jnp.sum(y, 0, keepdims=True),
         jnp.sum(y * y, 0, keepdims=True),
         jnp.zeros((6, cn), _F32)], axis=0)

    @pl.when(t == NSTEP - 1)
    def _():
        mu2 = acc[0:1, :] / E
        var = acc[1:2, :] / E - mu2 * mu2
        stn_ref[...] = jnp.concatenate([mu2, lax.rsqrt(var + 1e-5)], axis=0)


def _mlp0(vj, u, st, g, be, w, b, c, cv):
    return pl.pallas_call(
        _mlp0_body,
        grid=(NSTEP,),
        in_specs=[
            pl.BlockSpec((EB, cv), lambda t: (t, 0)),
            pl.BlockSpec((NB, c), lambda t: (t, 0)),
            pl.BlockSpec((2, c), lambda t: (0, 0)),
            pl.BlockSpec((1, c), lambda t: (0, 0)),
            pl.BlockSpec((1, c), lambda t: (0, 0)),
            pl.BlockSpec((c, c), lambda t: (0, 0)),
            pl.BlockSpec((1, c), lambda t: (0, 0)),
        ],
        out_specs=[
            pl.BlockSpec((EB, c), lambda t: (t, 0)),
            pl.BlockSpec((2, c), lambda t: (0, 0)),
        ],
        out_shape=[
            jax.ShapeDtypeStruct((E, c), _F32),
            jax.ShapeDtypeStruct((2, c), _F32),
        ],
        scratch_shapes=[pltpu.VMEM((8, c), _F32)],
    )(vj, u, st, g, be, w, b)


def _mlp1_body(y_ref, st_ref, g_ref, be_ref, w_ref, b_ref,
               yn_ref, stn_ref, acc):
    t = pl.program_id(0)
    mu = st_ref[0:1, :]
    rstd = st_ref[1:2, :]
    scale = g_ref[...] * rstd
    shift = be_ref[...] - mu * scale
    h = jnp.maximum(y_ref[...] * scale + shift, 0.0)
    y = _dot(h, w_ref[...]) + b_ref[...]
    yn_ref[...] = y

    @pl.when(t == 0)
    def _():
        acc[...] = jnp.zeros_like(acc)

    cn = y.shape[1]
    acc[...] += jnp.concatenate(
        [jnp.sum(y, 0, keepdims=True),
         jnp.sum(y * y, 0, keepdims=True),
         jnp.zeros((6, cn), _F32)], axis=0)

    @pl.when(t == NSTEP - 1)
    def _():
        mu2 = acc[0:1, :] / E
        var = acc[1:2, :] / E - mu2 * mu2
        stn_ref[...] = jnp.concatenate([mu2, lax.rsqrt(var + 1e-5)], axis=0)


def _mlp1(y, st, g, be, w, b, c):
    return pl.pallas_call(
        _mlp1_body,
        grid=(NSTEP,),
        in_specs=[
            pl.BlockSpec((EB, c), lambda t: (t, 0)),
            pl.BlockSpec((2, c), lambda t: (0, 0)),
            pl.BlockSpec((1, c), lambda t: (0, 0)),
            pl.BlockSpec((1, c), lambda t: (0, 0)),
            pl.BlockSpec((c, c), lambda t: (0, 0)),
            pl.BlockSpec((1, c), lambda t: (0, 0)),
        ],
        out_specs=[
            pl.BlockSpec((EB, c), lambda t: (t, 0)),
            pl.BlockSpec((2, c), lambda t: (0, 0)),
        ],
        out_shape=[
            jax.ShapeDtypeStruct((E, c), _F32),
            jax.ShapeDtypeStruct((2, c), _F32),
        ],
        scratch_shapes=[pltpu.VMEM((8, c), _F32)],
    )(y, st, g, be, w, b)


def _combine_body(y_ref, st_ref, g_ref, be_ref, s_ref, sst_ref,
                  scg_ref, scb_ref, xn_ref):
    mu = st_ref[0:1, :]
    rstd = st_ref[1:2, :]
    scale = g_ref[...] * rstd
    shift = be_ref[...] - mu * scale
    h = jnp.maximum(y_ref[...] * scale + shift, 0.0)
    c = h.shape[1]
    agg = jnp.mean(h.reshape(NB, K, c), axis=1)
    smu = sst_ref[0:1, :]
    srstd = sst_ref[1:2, :]
    sscale = scg_ref[...] * srstd
    sshift = scb_ref[...] - smu * sscale
    bs = s_ref[...] * sscale + sshift
    xn_ref[...] = jnp.maximum(agg + bs, 0.0)


def _combine(y2, st2, g, be, s, sst, scg, scb, c):
    return pl.pallas_call(
        _combine_body,
        grid=(NSTEP,),
        in_specs=[
            pl.BlockSpec((EB, c), lambda t: (t, 0)),
            pl.BlockSpec((2, c), lambda t: (0, 0)),
            pl.BlockSpec((1, c), lambda t: (0, 0)),
            pl.BlockSpec((1, c), lambda t: (0, 0)),
            pl.BlockSpec((NB, c), lambda t: (t, 0)),
            pl.BlockSpec((2, c), lambda t: (0, 0)),
            pl.BlockSpec((1, c), lambda t: (0, 0)),
            pl.BlockSpec((1, c), lambda t: (0, 0)),
        ],
        out_specs=pl.BlockSpec((NB, c), lambda t: (t, 0)),
        out_shape=jax.ShapeDtypeStruct((N, c), _F32),
    )(y2, st2, g, be, s, sst, scg, scb)


def _head_body(x_ref, b_ref, fcw_ref, fcb_ref, ow_ref, ob_ref,
               out_ref, accs, accc):
    t = pl.program_id(0)
    xb = x_ref[...]
    bb = b_ref[...].reshape(1, XB)
    gi = lax.broadcasted_iota(_I32, (104, XB), 0)
    m = (gi == bb).astype(_F32)

    @pl.when(t == 0)
    def _():
        accs[...] = jnp.zeros_like(accs)
        accc[...] = jnp.zeros_like(accc)

    accs[...] += _dot(m, xb)
    cnt = jnp.sum(m, axis=1, keepdims=True)
    accc[...] += jnp.broadcast_to(cnt, (104, 128))

    @pl.when(t == XSTEP - 1)
    def _():
        cnts = jnp.maximum(accc[:, 0:1], 1.0)
        pooled = accs[...] / cnts
        h = jnp.maximum(_dot(pooled, fcw_ref[...]) + fcb_ref[...], 0.0)
        out_ref[...] = _dot(h, ow_ref[...]) + ob_ref[...]


def _head(x, b3, fcw, fcb, ow, ob):
    return pl.pallas_call(
        _head_body,
        grid=(XSTEP,),
        in_specs=[
            pl.BlockSpec((XB, 256), lambda t: (t, 0)),
            pl.BlockSpec((1, 1, XB), lambda t: (t, 0, 0)),
            pl.BlockSpec((256, 256), lambda t: (0, 0)),
            pl.BlockSpec((1, 256), lambda t: (0, 0)),
            pl.BlockSpec((256, 16), lambda t: (0, 0)),
            pl.BlockSpec((1, 16), lambda t: (0, 0)),
        ],
        out_specs=pl.BlockSpec((104, 16), lambda t: (0, 0)),
        out_shape=jax.ShapeDtypeStruct((104, 16), _F32),
        scratch_shapes=[pltpu.VMEM((104, 256), _F32),
                        pltpu.VMEM((104, 128), _F32)],
    )(x, b3, fcw, fcb, ow, ob)


def kernel(x, batch,
           l0_w0, l0_b0, l0_g0, l0_be0,
           l0_w1, l0_b1, l0_g1, l0_be1,
           l0_w2, l0_b2, l0_g2, l0_be2,
           l0_scw, l0_scg, l0_scb,
           l1_w0, l1_b0, l1_g0, l1_be0,
           l1_w1, l1_b1, l1_g1, l1_be1,
           l1_w2, l1_b2, l1_g2, l1_be2,
           l1_scw, l1_scg, l1_scb,
           l2_w0, l2_b0, l2_g0, l2_be0,
           l2_w1, l2_b1, l2_g1, l2_be1,
           l2_w2, l2_b2, l2_g2, l2_be2,
           l2_scw, l2_scg, l2_scb,
           fc_w, fc_b, out_w, out_b):
    p = {
        0: (l0_w0, l0_b0, l0_g0, l0_be0, l0_w1, l0_b1, l0_g1, l0_be1,
            l0_w2, l0_b2, l0_g2, l0_be2, l0_scw, l0_scg, l0_scb),
        1: (l1_w0, l1_b0, l1_g0, l1_be0, l1_w1, l1_b1, l1_g1, l1_be1,
            l1_w2, l1_b2, l1_g2, l1_be2, l1_scw, l1_scg, l1_scb),
        2: (l2_w0, l2_b0, l2_g0, l2_be0, l2_w1, l2_b1, l2_g1, l2_be1,
            l2_w2, l2_b2, l2_g2, l2_be2, l2_scw, l2_scg, l2_scb),
    }
    cfgs = [(32, 64), (64, 128), (128, 256)]

    bi = batch.astype(_I32)
    bpad = jnp.pad(bi, (0, NPAD - N), constant_values=127)
    bp = bpad.reshape(1, NPAD)
    brow = bpad.reshape(NPAD, 1)
    b3 = bi.reshape(XSTEP, 1, XB)

    h = x
    for l, (f, c) in enumerate(cfgs):
        (w0, b0, g0, be0, w1, b1, g1, be1,
         w2, b2, g2, be2, scw, scg, scb) = p[l]
        cv = max(c, 128)
        xpad = jnp.pad(h, ((0, NPAD - N), (0, 0)))
        idx = _knn(xpad, bp, brow, bi, f)
        u, v, s, sst = _dense(h, w0, b0.reshape(1, c), scw, f, c, cv)
        idx3 = idx.reshape(_NW, _NCH, _CH)
        vj = _edge_gather(v, idx3, cv)
        st0 = _stats0(vj, u, c, cv)
        y1, st1 = _mlp0(vj, u, st0, g0.reshape(1, c), be0.reshape(1, c),
                        w1, b1.reshape(1, c), c, cv)
        y2, st2 = _mlp1(y1, st1, g1.reshape(1, c), be1.reshape(1, c),
                        w2, b2.reshape(1, c), c)
        h = _combine(y2, st2, g2.reshape(1, c), be2.reshape(1, c),
                     s, sst, scg.reshape(1, c), scb.reshape(1, c), c)

    ow = jnp.pad(out_w, ((0, 0), (0, 6)))
    ob = jnp.pad(out_b, (0, 6)).reshape(1, 16)
    logits = _head(h, b3, fc_w, fc_b.reshape(1, 256), ow, ob)
    return logits[:NG, :10]


# speedup vs baseline: 1.1941x; 1.1941x over previous
import functools

import jax
import jax.numpy as jnp
from jax import lax
from jax.experimental import pallas as pl
from jax.experimental.pallas import tpu as pltpu
from jax.experimental.pallas import tpu_sc as plsc

N = 10000
NG = 100
K = 16
E = N * K

KNN_R = 200
KNN_C = 512
NPAD = 10240

EB = 6400
NB = 400
NSTEP = E // EB

XB = 2000
XSTEP = N // XB

_F32 = jnp.float32
_I32 = jnp.int32
_INF = float("inf")


def _dot(a, b):
    return lax.dot_general(a, b, (((1,), (0,)), ((), ())),
                           preferred_element_type=_F32)


def _knn_body(xp_ref, bp_ref, brow_ref, bsm_ref, out_ref):
    t = pl.program_id(0)
    rs = t * KNN_R
    xr = xp_ref[pl.ds(rs, KNN_R), :]
    rb = brow_ref[pl.ds(rs, KNN_R), :]
    sqr = jnp.sum(xr * xr, axis=1, keepdims=True)

    bp = bp_ref[...]
    b0 = bsm_ref[rs]
    bl = bsm_ref[rs + KNN_R - 1]
    wstart = jnp.sum((bp < b0).astype(_I32))
    wend = jnp.sum((bp <= bl).astype(_I32))
    ws8 = jnp.minimum((wstart // 8) * 8, NPAD - KNN_C)

    rbt = rb.reshape(1, KNN_R)
    sqrt_ = sqr.reshape(1, KNN_R)
    rit = rs + lax.broadcasted_iota(_I32, (KNN_C, KNN_R), 1)
    cit = lax.broadcasted_iota(_I32, (KNN_C, KNN_R), 0)
    hc = KNN_C // 2
    big = 1 << 30

    def chunk(carry):
        k, tv, ti = carry
        cs = ws8 + k * KNN_C
        xc = xp_ref[pl.ds(cs, KNN_C), :]
        cb = brow_ref[pl.ds(cs, KNN_C), :]
        sqc = jnp.sum(xc * xc, axis=1, keepdims=True)
        dots = lax.dot_general(xc, xr, (((1,), (1,)), ((), ())),
                               preferred_element_type=_F32)
        d2 = sqc + sqrt_ - 2.0 * dots
        gci = cit + cs
        bad = (cb != rbt) | (rit == gci)
        d2 = jnp.where(bad, _INF, d2)

        a, b = d2[:hc, :], d2[hc:, :]
        ia, ib = gci[:hc, :], gci[hc:, :]
        sel = a <= b
        lo = jnp.where(sel, a, b)
        hi = jnp.where(sel, b, a)
        li = jnp.where(sel, ia, ib)
        hj = jnp.where(sel, ib, ia)

        cv = jnp.concatenate([tv, lo], axis=0)
        cidx = jnp.concatenate([ti, li], axis=0)
        sib = jnp.concatenate([jnp.full((K, KNN_R), _INF, _F32), hi], axis=0)
        sidx = jnp.concatenate([ti, hj], axis=0)
        vs, js = [], []
        for _ in range(K):
            m = jnp.min(cv, axis=0, keepdims=True)
            elig = cv == m
            idxk = jnp.min(jnp.where(elig, cidx, big), axis=0, keepdims=True)
            kill = elig & (cidx == idxk)
            vs.append(m)
            js.append(idxk)
            cv = jnp.where(kill, sib, cv)
            cidx = jnp.where(kill, sidx, cidx)
            sib = jnp.where(kill, _INF, sib)
        tv = jnp.concatenate(vs, axis=0)
        ti = jnp.concatenate(js, axis=0)
        return k + 1, tv, ti

    def cond(carry):
        return ws8 + carry[0] * KNN_C < wend

    tv0 = jnp.full((K, KNN_R), _INF, _F32)
    ti0 = jnp.zeros((K, KNN_R), _I32)
    _, _, ti = lax.while_loop(cond, chunk, (0, tv0, ti0))
    out_ref[...] = jnp.minimum(ti.T, N - 1)


def _knn(x, bp, brow, bsm, f):
    return pl.pallas_call(
        _knn_body,
        grid=(N // KNN_R,),
        in_specs=[
            pl.BlockSpec((NPAD, f), lambda t: (0, 0)),
            pl.BlockSpec((1, NPAD), lambda t: (0, 0)),
            pl.BlockSpec((NPAD, 1), lambda t: (0, 0)),
            pl.BlockSpec(memory_space=pltpu.SMEM),
        ],
        out_specs=pl.BlockSpec((KNN_R, K), lambda t: (t, 0)),
        out_shape=jax.ShapeDtypeStruct((N, K), _I32),
    )(x, bp, brow, bsm)


def _dense_body(f, cv, x_ref, w0_ref, b0_ref, scw_ref,
                u_ref, v_ref, s_ref, st_ref, acc):
    t = pl.program_id(0)
    xb = x_ref[...]
    w0 = w0_ref[...]
    wa = w0[:f, :]
    wb = w0[f:, :]
    c = w0.shape[1]
    u_ref[...] = _dot(xb, wa - wb) + b0_ref[...]
    v = _dot(xb, wb)
    if cv > c:
        v = jnp.concatenate([v, jnp.zeros((XB, cv - c), _F32)], axis=1)
    v_ref[...] = v
    s = _dot(xb, scw_ref[...])
    s_ref[...] = s

    @pl.when(t == 0)
    def _():
        acc[...] = jnp.zeros_like(acc)

    c = s.shape[1]
    upd = jnp.concatenate(
        [jnp.sum(s, 0, keepdims=True),
         jnp.sum(s * s, 0, keepdims=True),
         jnp.zeros((6, c), _F32)], axis=0)
    acc[...] += upd

    @pl.when(t == XSTEP - 1)
    def _():
        mu = acc[0:1, :] / N
        var = acc[1:2, :] / N - mu * mu
        st_ref[...] = jnp.concatenate([mu, lax.rsqrt(var + 1e-5)], axis=0)


def _dense(x, w0, b0, scw, f, c, cv):
    return pl.pallas_call(
        functools.partial(_dense_body, f, cv),
        grid=(XSTEP,),
        in_specs=[
            pl.BlockSpec((XB, f), lambda t: (t, 0)),
            pl.BlockSpec((2 * f, c), lambda t: (0, 0)),
            pl.BlockSpec((1, c), lambda t: (0, 0)),
            pl.BlockSpec((f, c), lambda t: (0, 0)),
        ],
        out_specs=[
            pl.BlockSpec((XB, c), lambda t: (t, 0)),
            pl.BlockSpec((XB, cv), lambda t: (t, 0)),
            pl.BlockSpec((XB, c), lambda t: (t, 0)),
            pl.BlockSpec((2, c), lambda t: (0, 0)),
        ],
        out_shape=[
            jax.ShapeDtypeStruct((N, c), _F32),
            jax.ShapeDtypeStruct((N, cv), _F32),
            jax.ShapeDtypeStruct((N, c), _F32),
            jax.ShapeDtypeStruct((2, c), _F32),
        ],
        scratch_shapes=[pltpu.VMEM((8, c), _F32)],
    )(x, w0, b0, scw)


_NC = 2
_NS = 16
_NW = _NC * _NS
_PERW = E // _NW
_CH = 40
_NCH = _PERW // _CH


def _edge_gather(v, idx3, c):
    mesh = plsc.VectorSubcoreMesh(core_axis_name="c", subcore_axis_name="s")

    @functools.partial(
        pl.kernel, mesh=mesh,
        out_type=jax.ShapeDtypeStruct((E, c), _F32),
        scratch_types=[
            pltpu.VMEM((_NCH, _CH), _I32),
            pltpu.VMEM((_CH, c), _F32),
            pltpu.VMEM((_CH, c), _F32),
            pltpu.SemaphoreType.DMA,
            pltpu.SemaphoreType.DMA,
        ],
    )
    def gk(v_hbm, idx_hbm, out_hbm, idx_v, buf0, buf1, sem0, sem1):
        wid = lax.axis_index("s") * _NC + lax.axis_index("c")
        base = wid * _PERW
        pltpu.sync_copy(idx_hbm.at[wid], idx_v)

        def wait0():
            pltpu.make_async_copy(v_hbm.at[idx_v.at[0]], buf0, sem0).wait()

        def wait1():
            pltpu.make_async_copy(v_hbm.at[idx_v.at[0]], buf1, sem1).wait()

        pltpu.async_copy(v_hbm.at[idx_v.at[0]], buf0, sem0)

        def step(j, carry):
            c0 = 2 * j
            pltpu.async_copy(v_hbm.at[idx_v.at[c0 + 1]], buf1, sem1)
            wait0()
            pltpu.sync_copy(buf0, out_hbm.at[pl.ds(base + c0 * _CH, _CH)])
            pltpu.async_copy(v_hbm.at[idx_v.at[c0 + 2]], buf0, sem0)
            wait1()
            pltpu.sync_copy(buf1,
                            out_hbm.at[pl.ds(base + (c0 + 1) * _CH, _CH)])
            return carry

        lax.fori_loop(0, (_NCH - 1) // 2, step, 0, unroll=False)
        wait0()
        pltpu.sync_copy(buf0,
                        out_hbm.at[pl.ds(base + (_NCH - 1) * _CH, _CH)])

    return gk(v, idx3)


def _stats0_body(vj_ref, u_ref, st_ref, acc):
    t = pl.program_id(0)
    u = u_ref[...]
    c = u.shape[1]
    urep = jnp.broadcast_to(u[:, None, :], (NB, K, c)).reshape(EB, c)
    y = vj_ref[:, :c] + urep

    @pl.when(t == 0)
    def _():
        acc[...] = jnp.zeros_like(acc)

    acc[...] += jnp.concatenate(
        [jnp.sum(y, 0, keepdims=True),
         jnp.sum(y * y, 0, keepdims=True),
         jnp.zeros((6, c), _F32)], axis=0)

    @pl.when(t == NSTEP - 1)
    def _():
        mu = acc[0:1, :] / E
        var = acc[1:2, :] / E - mu * mu
        st_ref[...] = jnp.concatenate([mu, lax.rsqrt(var + 1e-5)], axis=0)


def _stats0(vj, u, c, cv):
    return pl.pallas_call(
        _stats0_body,
        grid=(NSTEP,),
        in_specs=[
            pl.BlockSpec((EB, cv), lambda t: (t, 0)),
            pl.BlockSpec((NB, c), lambda t: (t, 0)),
        ],
        out_specs=pl.BlockSpec((2, c), lambda t: (0, 0)),
        out_shape=jax.ShapeDtypeStruct((2, c), _F32),
        scratch_shapes=[pltpu.VMEM((8, c), _F32)],
    )(vj, u)


def _mlp0_body(vj_ref, u_ref, st_ref, g_ref, be_ref, w_ref, b_ref,
               y_ref, stn_ref, acc):
    t = pl.program_id(0)
    u = u_ref[...]
    c = u.shape[1]
    urep = jnp.broadcast_to(u[:, None, :], (NB, K, c)).reshape(EB, c)
    y0 = vj_ref[:, :c] + urep
    mu = st_ref[0:1, :]
    rstd = st_ref[1:2, :]
    scale = g_ref[...] * rstd
    shift = be_ref[...] - mu * scale
    h = jnp.maximum(y0 * scale + shift, 0.0)
    y = _dot(h, w_ref[...]) + b_ref[...]
    y_ref[...] = y

    @pl.when(t == 0)
    def _():
        acc[...] = jnp.zeros_like(acc)

    cn = y.shape[1]
    acc[...] += jnp.concatenate(
        [jnp.sum(y, 0, keepdims=True),
         jnp.sum(y * y, 0, keepdims=True),
         jnp.zeros((6, cn), _F32)], axis=0)

    @pl.when(t == NSTEP - 1)
    def _():
        mu2 = acc[0:1, :] / E
        var = acc[1:2, :] / E - mu2 * mu2
        stn_ref[...] = jnp.concatenate([mu2, lax.rsqrt(var + 1e-5)], axis=0)


def _mlp0(vj, u, st, g, be, w, b, c, cv):
    return pl.pallas_call(
        _mlp0_body,
        grid=(NSTEP,),
        in_specs=[
            pl.BlockSpec((EB, cv), lambda t: (t, 0)),
            pl.BlockSpec((NB, c), lambda t: (t, 0)),
            pl.BlockSpec((2, c), lambda t: (0, 0)),
            pl.BlockSpec((1, c), lambda t: (0, 0)),
            pl.BlockSpec((1, c), lambda t: (0, 0)),
            pl.BlockSpec((c, c), lambda t: (0, 0)),
            pl.BlockSpec((1, c), lambda t: (0, 0)),
        ],
        out_specs=[
            pl.BlockSpec((EB, c), lambda t: (t, 0)),
            pl.BlockSpec((2, c), lambda t: (0, 0)),
        ],
        out_shape=[
            jax.ShapeDtypeStruct((E, c), _F32),
            jax.ShapeDtypeStruct((2, c), _F32),
        ],
        scratch_shapes=[pltpu.VMEM((8, c), _F32)],
    )(vj, u, st, g, be, w, b)


def _mlp1_body(y_ref, st_ref, g_ref, be_ref, w_ref, b_ref,
               yn_ref, stn_ref, acc):
    t = pl.program_id(0)
    mu = st_ref[0:1, :]
    rstd = st_ref[1:2, :]
    scale = g_ref[...] * rstd
    shift = be_ref[...] - mu * scale
    h = jnp.maximum(y_ref[...] * scale + shift, 0.0)
    y = _dot(h, w_ref[...]) + b_ref[...]
    yn_ref[...] = y

    @pl.when(t == 0)
    def _():
        acc[...] = jnp.zeros_like(acc)

    cn = y.shape[1]
    acc[...] += jnp.concatenate(
        [jnp.sum(y, 0, keepdims=True),
         jnp.sum(y * y, 0, keepdims=True),
         jnp.zeros((6, cn), _F32)], axis=0)

    @pl.when(t == NSTEP - 1)
    def _():
        mu2 = acc[0:1, :] / E
        var = acc[1:2, :] / E - mu2 * mu2
        stn_ref[...] = jnp.concatenate([mu2, lax.rsqrt(var + 1e-5)], axis=0)


def _mlp1(y, st, g, be, w, b, c):
    return pl.pallas_call(
        _mlp1_body,
        grid=(NSTEP,),
        in_specs=[
            pl.BlockSpec((EB, c), lambda t: (t, 0)),
            pl.BlockSpec((2, c), lambda t: (0, 0)),
            pl.BlockSpec((1, c), lambda t: (0, 0)),
            pl.BlockSpec((1, c), lambda t: (0, 0)),
            pl.BlockSpec((c, c), lambda t: (0, 0)),
            pl.BlockSpec((1, c), lambda t: (0, 0)),
        ],
        out_specs=[
            pl.BlockSpec((EB, c), lambda t: (t, 0)),
            pl.BlockSpec((2, c), lambda t: (0, 0)),
        ],
        out_shape=[
            jax.ShapeDtypeStruct((E, c), _F32),
            jax.ShapeDtypeStruct((2, c), _F32),
        ],
        scratch_shapes=[pltpu.VMEM((8, c), _F32)],
    )(y, st, g, be, w, b)


def _combine_body(y_ref, st_ref, g_ref, be_ref, s_ref, sst_ref,
                  scg_ref, scb_ref, xn_ref):
    mu = st_ref[0:1, :]
    rstd = st_ref[1:2, :]
    scale = g_ref[...] * rstd
    shift = be_ref[...] - mu * scale
    h = jnp.maximum(y_ref[...] * scale + shift, 0.0)
    c = h.shape[1]
    agg = jnp.mean(h.reshape(NB, K, c), axis=1)
    smu = sst_ref[0:1, :]
    srstd = sst_ref[1:2, :]
    sscale = scg_ref[...] * srstd
    sshift = scb_ref[...] - smu * sscale
    bs = s_ref[...] * sscale + sshift
    xn_ref[...] = jnp.maximum(agg + bs, 0.0)


def _combine(y2, st2, g, be, s, sst, scg, scb, c):
    return pl.pallas_call(
        _combine_body,
        grid=(NSTEP,),
        in_specs=[
            pl.BlockSpec((EB, c), lambda t: (t, 0)),
            pl.BlockSpec((2, c), lambda t: (0, 0)),
            pl.BlockSpec((1, c), lambda t: (0, 0)),
            pl.BlockSpec((1, c), lambda t: (0, 0)),
            pl.BlockSpec((NB, c), lambda t: (t, 0)),
            pl.BlockSpec((2, c), lambda t: (0, 0)),
            pl.BlockSpec((1, c), lambda t: (0, 0)),
            pl.BlockSpec((1, c), lambda t: (0, 0)),
        ],
        out_specs=pl.BlockSpec((NB, c), lambda t: (t, 0)),
        out_shape=jax.ShapeDtypeStruct((N, c), _F32),
    )(y2, st2, g, be, s, sst, scg, scb)


def _head_body(x_ref, b_ref, fcw_ref, fcb_ref, ow_ref, ob_ref,
               out_ref, accs, accc):
    t = pl.program_id(0)
    xb = x_ref[...]
    bb = b_ref[...].reshape(1, XB)
    gi = lax.broadcasted_iota(_I32, (104, XB), 0)
    m = (gi == bb).astype(_F32)

    @pl.when(t == 0)
    def _():
        accs[...] = jnp.zeros_like(accs)
        accc[...] = jnp.zeros_like(accc)

    accs[...] += _dot(m, xb)
    cnt = jnp.sum(m, axis=1, keepdims=True)
    accc[...] += jnp.broadcast_to(cnt, (104, 128))

    @pl.when(t == XSTEP - 1)
    def _():
        cnts = jnp.maximum(accc[:, 0:1], 1.0)
        pooled = accs[...] / cnts
        h = jnp.maximum(_dot(pooled, fcw_ref[...]) + fcb_ref[...], 0.0)
        out_ref[...] = _dot(h, ow_ref[...]) + ob_ref[...]


def _head(x, b3, fcw, fcb, ow, ob):
    return pl.pallas_call(
        _head_body,
        grid=(XSTEP,),
        in_specs=[
            pl.BlockSpec((XB, 256), lambda t: (t, 0)),
            pl.BlockSpec((1, 1, XB), lambda t: (t, 0, 0)),
            pl.BlockSpec((256, 256), lambda t: (0, 0)),
            pl.BlockSpec((1, 256), lambda t: (0, 0)),
            pl.BlockSpec((256, 16), lambda t: (0, 0)),
            pl.BlockSpec((1, 16), lambda t: (0, 0)),
        ],
        out_specs=pl.BlockSpec((104, 16), lambda t: (0, 0)),
        out_shape=jax.ShapeDtypeStruct((104, 16), _F32),
        scratch_shapes=[pltpu.VMEM((104, 256), _F32),
                        pltpu.VMEM((104, 128), _F32)],
    )(x, b3, fcw, fcb, ow, ob)


def kernel(x, batch,
           l0_w0, l0_b0, l0_g0, l0_be0,
           l0_w1, l0_b1, l0_g1, l0_be1,
           l0_w2, l0_b2, l0_g2, l0_be2,
           l0_scw, l0_scg, l0_scb,
           l1_w0, l1_b0, l1_g0, l1_be0,
           l1_w1, l1_b1, l1_g1, l1_be1,
           l1_w2, l1_b2, l1_g2, l1_be2,
           l1_scw, l1_scg, l1_scb,
           l2_w0, l2_b0, l2_g0, l2_be0,
           l2_w1, l2_b1, l2_g1, l2_be1,
           l2_w2, l2_b2, l2_g2, l2_be2,
           l2_scw, l2_scg, l2_scb,
           fc_w, fc_b, out_w, out_b):
    p = {
        0: (l0_w0, l0_b0, l0_g0, l0_be0, l0_w1, l0_b1, l0_g1, l0_be1,
            l0_w2, l0_b2, l0_g2, l0_be2, l0_scw, l0_scg, l0_scb),
        1: (l1_w0, l1_b0, l1_g0, l1_be0, l1_w1, l1_b1, l1_g1, l1_be1,
            l1_w2, l1_b2, l1_g2, l1_be2, l1_scw, l1_scg, l1_scb),
        2: (l2_w0, l2_b0, l2_g0, l2_be0, l2_w1, l2_b1, l2_g1, l2_be1,
            l2_w2, l2_b2, l2_g2, l2_be2, l2_scw, l2_scg, l2_scb),
    }
    cfgs = [(32, 64), (64, 128), (128, 256)]

    bi = batch.astype(_I32)
    bpad = jnp.pad(bi, (0, NPAD - N), constant_values=127)
    bp = bpad.reshape(1, NPAD)
    brow = bpad.reshape(NPAD, 1)
    b3 = bi.reshape(XSTEP, 1, XB)

    h = x
    for l, (f, c) in enumerate(cfgs):
        (w0, b0, g0, be0, w1, b1, g1, be1,
         w2, b2, g2, be2, scw, scg, scb) = p[l]
        cv = max(c, 128)
        xpad = jnp.pad(h, ((0, NPAD - N), (0, 0)))
        idx = _knn(xpad, bp, brow, bi, f)
        u, v, s, sst = _dense(h, w0, b0.reshape(1, c), scw, f, c, cv)
        idx3 = idx.reshape(_NW, _NCH, _CH)
        vj = _edge_gather(v, idx3, cv)
        st0 = _stats0(vj, u, c, cv)
        y1, st1 = _mlp0(vj, u, st0, g0.reshape(1, c), be0.reshape(1, c),
                        w1, b1.reshape(1, c), c, cv)
        y2, st2 = _mlp1(y1, st1, g1.reshape(1, c), be1.reshape(1, c),
                        w2, b2.reshape(1, c), c)
        h = _combine(y2, st2, g2.reshape(1, c), be2.reshape(1, c),
                     s, sst, scg.reshape(1, c), scb.reshape(1, c), c)

    ow = jnp.pad(out_w, ((0, 0), (0, 6)))
    ob = jnp.pad(out_b, (0, 6)).reshape(1, 16)
    logits = _head(h, b3, fc_w, fc_b.reshape(1, 256), ow, ob)
    return logits[:NG, :10]
